# baseline (speedup 1.0000x reference)
import functools

import jax
import jax.numpy as jnp
from jax import lax
from jax.experimental import pallas as pl
from jax.experimental.pallas import tpu as pltpu
from jax.experimental.pallas import tpu_sc as plsc

N = 50000
FP = 1778
E = 3200000

NW = 32
CHUNK = 2048
CROWS = CHUNK // 128
NCH = E // CHUNK
CPT = NCH // NW
XTRA = NCH - CPT * NW
REM = NCH * CHUNK
RROWS = (E - REM) // 128
N_PAD = 50176
SLICE = N_PAD // 16

_mesh = plsc.VectorSubcoreMesh(core_axis_name="c", subcore_axis_name="s")


@functools.partial(
    pl.kernel,
    mesh=_mesh,
    out_type=jax.ShapeDtypeStruct((4 * N_PAD,), jnp.float32),
    compiler_params=pltpu.CompilerParams(needs_layout_passes=False),
    scratch_types=[
        pltpu.VMEM((2 * N_PAD,), jnp.float32),
        pltpu.VMEM((2, CHUNK), jnp.int32),
        pltpu.VMEM((2, CROWS, 128), jnp.int32),
        pltpu.VMEM((2, CHUNK), jnp.float32),
        pltpu.VMEM((2, CHUNK), jnp.float32),
        pltpu.VMEM((SLICE,), jnp.float32),
        pltpu.VMEM_SHARED((N_PAD,), jnp.float32),
        pltpu.VMEM_SHARED((N_PAD,), jnp.float32),
        pltpu.SemaphoreType.DMA,
        pltpu.SemaphoreType.DMA,
    ],
)
def _segment_sum_sc(xx, srch, dsth, zz, out, xxv, srcv, dstv, v0, v1, zbv,
                    acc0, acc1, sem_i, sem_s):
    cid = lax.axis_index("c")
    sid = lax.axis_index("s")
    wid = sid * 2 + cid

    pltpu.sync_copy(xx, xxv)

    pltpu.sync_copy(zz, zbv)
    pltpu.sync_copy(zbv, acc0.at[pl.ds(sid * SLICE, SLICE)])
    pltpu.sync_copy(zbv, acc1.at[pl.ds(sid * SLICE, SLICE)])
    plsc.subcore_barrier()

    nch = jnp.where(wid < XTRA, CPT + 1, CPT)
    ch0 = wid * CPT + jnp.minimum(wid, XTRA)

    def _gather(slot, n16):
        def _g(i, c2):
            s16 = srcv[slot, pl.ds(i * 16, 16)]
            v0[slot, pl.ds(i * 16, 16)] = plsc.load_gather(xxv, [s16])
            v1[slot, pl.ds(i * 16, 16)] = plsc.load_gather(xxv,
                                                           [s16 + N_PAD])
            return c2
        lax.fori_loop(0, n16, _g, 0)

    def _drain_scatters():
        for _ in range(4):
            pltpu.make_async_copy(xx.at[pl.ds(0, 1024)],
                                  zbv.at[pl.ds(0, 1024)], sem_s).wait()

    for _ in range(4):
        pltpu.async_copy(xx.at[pl.ds(0, 1024)], zbv.at[pl.ds(0, 1024)],
                         sem_s)
    pltpu.async_copy(srch.at[pl.ds(ch0 * CHUNK, CHUNK)], srcv.at[0], sem_i)
    pltpu.async_copy(dsth.at[pl.ds(ch0 * CROWS, CROWS)], dstv.at[0], sem_i)

    def _chunk(k, carry):
        p = lax.rem(k, 2)
        q = 1 - p
        _drain_scatters()
        pltpu.make_async_copy(srch.at[pl.ds(0, CHUNK)], srcv.at[p],
                              sem_i).wait()
        pltpu.make_async_copy(dsth.at[pl.ds(0, CROWS)], dstv.at[p],
                              sem_i).wait()
        c2 = jnp.minimum(ch0 + k + 1, NCH - 1)
        pltpu.async_copy(srch.at[pl.ds(c2 * CHUNK, CHUNK)], srcv.at[q],
                         sem_i)
        pltpu.async_copy(dsth.at[pl.ds(c2 * CROWS, CROWS)], dstv.at[q],
                         sem_i)
        _gather(p, CHUNK // 16)
        for j in range(CROWS):
            pltpu.async_copy(v0.at[p].at[pl.ds(j * 128, 128)],
                             acc0.at[dstv.at[p, j]], sem_s, add=True)
            pltpu.async_copy(v1.at[p].at[pl.ds(j * 128, 128)],
                             acc1.at[dstv.at[p, j]], sem_s, add=True)
        return carry
    lax.fori_loop(0, nch, _chunk, 0)

    _drain_scatters()
    pltpu.make_async_copy(srch.at[pl.ds(0, CHUNK)], srcv.at[0], sem_i).wait()
    pltpu.make_async_copy(dsth.at[pl.ds(0, CROWS)], dstv.at[0], sem_i).wait()

    @pl.when(wid == NW - 1)
    def _rem():
        pltpu.sync_copy(srch.at[pl.ds(REM, 128 * RROWS)],
                        srcv.at[0].at[pl.ds(0, 128 * RROWS)])
        pltpu.sync_copy(dsth.at[pl.ds(NCH * CROWS, RROWS)],
                        dstv.at[0].at[pl.ds(0, RROWS)])
        _gather(0, (128 * RROWS) // 16)
        for j in range(RROWS):
            pltpu.sync_copy(v0.at[0].at[pl.ds(j * 128, 128)],
                            acc0.at[dstv.at[0, j]], add=True)
            pltpu.sync_copy(v1.at[0].at[pl.ds(j * 128, 128)],
                            acc1.at[dstv.at[0, j]], add=True)

    plsc.subcore_barrier()
    pltpu.sync_copy(acc0.at[pl.ds(sid * SLICE, SLICE)], zbv)
    pltpu.sync_copy(zbv, out.at[pl.ds(cid * 2 * N_PAD + sid * SLICE, SLICE)])
    pltpu.sync_copy(acc1.at[pl.ds(sid * SLICE, SLICE)], zbv)
    pltpu.sync_copy(zbv, out.at[pl.ds((cid * 2 + 1) * N_PAD + sid * SLICE,
                                      SLICE)])


def _affine_body(p_ref, xx_ref, hw_ref, hb_ref, o_ref):
    v = p_ref[0] + p_ref[1] + xx_ref[...]
    hw = hw_ref[...]
    z = (hw[:, 0:1] * v[0:1, :] + hw[:, 1:2] * v[1:2, :]) + hb_ref[...]
    o_ref[...] = 1.0 / (1.0 + jnp.exp(-z))


def _affine_sigmoid(p, xx, H_w, H_b):
    return pl.pallas_call(
        _affine_body,
        out_shape=jax.ShapeDtypeStruct((2, N_PAD), jnp.float32),
    )(p, xx, H_w, H_b.reshape(2, 1))


R = 1024
LOG2E = 1.4426950408889634


def _fp_body(a1_ref, p2_ref, h2w_ref, h2b_ref, w1c_ref, w2c_ref, o_ref):
    a1 = a1_ref[...]
    v2 = p2_ref[0] + p2_ref[1] + a1
    hw = h2w_ref[...]
    z2 = (hw[:, 0:1] * v2[0:1, :] + hw[:, 1:2] * v2[1:2, :]) + h2b_ref[...]
    a2 = 1.0 / (1.0 + jnp.exp(-z2))

    onesr = jnp.ones((1, R), jnp.float32)
    ones = jnp.ones((1, FP), jnp.float32)

    def _soft(a, wc_ref):
        av = jnp.concatenate([a, onesr], axis=0)
        l = jnp.dot(wc_ref[...], av,
                    preferred_element_type=jnp.float32)
        e = jnp.exp2(l)
        s = jnp.dot(ones, e, preferred_element_type=jnp.float32)
        return e * (1.0 / s)

    o_ref[...] = _soft(a1, w1c_ref) + _soft(a2, w2c_ref)


def _fingerprint(a1, p2, H2_w, H2_b, W1_w, W1_b, W2_w, W2_b):
    w1c = jnp.concatenate([W1_w, W1_b.reshape(FP, 1)], axis=1) * LOG2E
    w2c = jnp.concatenate([W2_w, W2_b.reshape(FP, 1)], axis=1) * LOG2E
    return pl.pallas_call(
        _fp_body,
        grid=(N_PAD // R,),
        in_specs=[
            pl.BlockSpec((2, R), lambda i: (0, i)),
            pl.BlockSpec((2, 2, R), lambda i: (0, 0, i)),
            pl.BlockSpec((2, 2), lambda i: (0, 0)),
            pl.BlockSpec((2, 1), lambda i: (0, 0)),
            pl.BlockSpec((FP, 3), lambda i: (0, 0)),
            pl.BlockSpec((FP, 3), lambda i: (0, 0)),
        ],
        out_specs=pl.BlockSpec((FP, R), lambda i: (0, i)),
        out_shape=jax.ShapeDtypeStruct((FP, N), jnp.float32),
    )(a1, p2, H2_w, H2_b.reshape(2, 1), w1c, w2c)


def kernel(x, edge_index, H1_w, H1_b, W1_w, W1_b, H2_w, H2_b, W2_w, W2_b):
    ei = edge_index.astype(jnp.int32)
    eif = ei.reshape(-1)
    dst2d = ei[1].reshape(E // 128, 128)
    xx = jnp.zeros((2, N_PAD), jnp.float32).at[:, :N].set(x.T)
    zz = jnp.zeros((SLICE,), jnp.float32)

    p1 = _segment_sum_sc(xx.reshape(-1), eif, dst2d, zz).reshape(2, 2, N_PAD)
    a1 = _affine_sigmoid(p1, xx, H1_w, H1_b)
    p2 = _segment_sum_sc(a1.reshape(-1), eif, dst2d, zz).reshape(2, 2, N_PAD)

    return _fingerprint(a1, p2, H2_w, H2_b, W1_w, W1_b, W2_w, W2_b).T


# speedup vs baseline: 1.0250x; 1.0250x over previous
import functools

import jax
import jax.numpy as jnp
from jax import lax
from jax.experimental import pallas as pl
from jax.experimental.pallas import tpu as pltpu
from jax.experimental.pallas import tpu_sc as plsc

N = 50000
FP = 1778
E = 3200000

NW = 32
CHUNK = 2048
CROWS = CHUNK // 128
NCH = E // CHUNK
CPT = NCH // NW
XTRA = NCH - CPT * NW
REM = NCH * CHUNK
RROWS = (E - REM) // 128
N_PAD = 50176
SLICE = N_PAD // 16

_mesh = plsc.VectorSubcoreMesh(core_axis_name="c", subcore_axis_name="s")


@functools.partial(
    pl.kernel,
    mesh=_mesh,
    out_type=jax.ShapeDtypeStruct((4 * N_PAD,), jnp.float32),
    compiler_params=pltpu.CompilerParams(needs_layout_passes=False),
    scratch_types=[
        pltpu.VMEM((2 * N_PAD,), jnp.float32),
        pltpu.VMEM((2, CHUNK), jnp.int32),
        pltpu.VMEM((2, CROWS, 128), jnp.int32),
        pltpu.VMEM((2, CHUNK), jnp.float32),
        pltpu.VMEM((2, CHUNK), jnp.float32),
        pltpu.VMEM((SLICE,), jnp.float32),
        pltpu.VMEM_SHARED((N_PAD,), jnp.float32),
        pltpu.VMEM_SHARED((N_PAD,), jnp.float32),
        pltpu.SemaphoreType.DMA,
        pltpu.SemaphoreType.DMA,
    ],
)
def _segment_sum_sc(xx, srch, dsth, zz, out, xxv, srcv, dstv, v0, v1, zbv,
                    acc0, acc1, sem_i, sem_s):
    cid = lax.axis_index("c")
    sid = lax.axis_index("s")
    wid = sid * 2 + cid

    pltpu.sync_copy(xx, xxv)

    pltpu.sync_copy(zz, zbv)
    pltpu.sync_copy(zbv, acc0.at[pl.ds(sid * SLICE, SLICE)])
    pltpu.sync_copy(zbv, acc1.at[pl.ds(sid * SLICE, SLICE)])
    plsc.subcore_barrier()

    nch = jnp.where(wid < XTRA, CPT + 1, CPT)
    ch0 = wid * CPT + jnp.minimum(wid, XTRA)

    def _gather(slot, n16):
        def _g(i, c2):
            s16 = srcv[slot, pl.ds(i * 16, 16)]
            v0[slot, pl.ds(i * 16, 16)] = plsc.load_gather(xxv, [s16])
            v1[slot, pl.ds(i * 16, 16)] = plsc.load_gather(xxv,
                                                           [s16 + N_PAD])
            return c2
        lax.fori_loop(0, n16, _g, 0)

    def _drain_scatters():
        for _ in range(4):
            pltpu.make_async_copy(xx.at[pl.ds(0, 1024)],
                                  zbv.at[pl.ds(0, 1024)], sem_s).wait()

    for _ in range(4):
        pltpu.async_copy(xx.at[pl.ds(0, 1024)], zbv.at[pl.ds(0, 1024)],
                         sem_s)
    pltpu.async_copy(srch.at[pl.ds(ch0 * CHUNK, CHUNK)], srcv.at[0], sem_i)
    pltpu.async_copy(dsth.at[pl.ds(ch0 * CROWS, CROWS)], dstv.at[0], sem_i)

    def _chunk(k, carry):
        p = lax.rem(k, 2)
        q = 1 - p
        _drain_scatters()
        pltpu.make_async_copy(srch.at[pl.ds(0, CHUNK)], srcv.at[p],
                              sem_i).wait()
        pltpu.make_async_copy(dsth.at[pl.ds(0, CROWS)], dstv.at[p],
                              sem_i).wait()
        c2 = jnp.minimum(ch0 + k + 1, NCH - 1)
        pltpu.async_copy(srch.at[pl.ds(c2 * CHUNK, CHUNK)], srcv.at[q],
                         sem_i)
        pltpu.async_copy(dsth.at[pl.ds(c2 * CROWS, CROWS)], dstv.at[q],
                         sem_i)
        _gather(p, CHUNK // 16)
        for j in range(CROWS):
            pltpu.async_copy(v0.at[p].at[pl.ds(j * 128, 128)],
                             acc0.at[dstv.at[p, j]], sem_s, add=True)
            pltpu.async_copy(v1.at[p].at[pl.ds(j * 128, 128)],
                             acc1.at[dstv.at[p, j]], sem_s, add=True)
        return carry
    lax.fori_loop(0, nch, _chunk, 0)

    _drain_scatters()
    pltpu.make_async_copy(srch.at[pl.ds(0, CHUNK)], srcv.at[0], sem_i).wait()
    pltpu.make_async_copy(dsth.at[pl.ds(0, CROWS)], dstv.at[0], sem_i).wait()

    @pl.when(wid == NW - 1)
    def _rem():
        pltpu.sync_copy(srch.at[pl.ds(REM, 128 * RROWS)],
                        srcv.at[0].at[pl.ds(0, 128 * RROWS)])
        pltpu.sync_copy(dsth.at[pl.ds(NCH * CROWS, RROWS)],
                        dstv.at[0].at[pl.ds(0, RROWS)])
        _gather(0, (128 * RROWS) // 16)
        for j in range(RROWS):
            pltpu.sync_copy(v0.at[0].at[pl.ds(j * 128, 128)],
                            acc0.at[dstv.at[0, j]], add=True)
            pltpu.sync_copy(v1.at[0].at[pl.ds(j * 128, 128)],
                            acc1.at[dstv.at[0, j]], add=True)

    plsc.subcore_barrier()
    pltpu.sync_copy(acc0.at[pl.ds(sid * SLICE, SLICE)], zbv)
    pltpu.sync_copy(zbv, out.at[pl.ds(cid * 2 * N_PAD + sid * SLICE, SLICE)])
    pltpu.sync_copy(acc1.at[pl.ds(sid * SLICE, SLICE)], zbv)
    pltpu.sync_copy(zbv, out.at[pl.ds((cid * 2 + 1) * N_PAD + sid * SLICE,
                                      SLICE)])


def _affine_body(p_ref, xx_ref, hw_ref, hb_ref, o_ref):
    v = p_ref[0] + p_ref[1] + xx_ref[...]
    hw = hw_ref[...]
    z = (hw[:, 0:1] * v[0:1, :] + hw[:, 1:2] * v[1:2, :]) + hb_ref[...]
    o_ref[...] = 1.0 / (1.0 + jnp.exp(-z))


def _affine_sigmoid(p, xx, H_w, H_b):
    return pl.pallas_call(
        _affine_body,
        out_shape=jax.ShapeDtypeStruct((2, N_PAD), jnp.float32),
    )(p, xx, H_w, H_b.reshape(2, 1))


R = 1024
LOG2E = 1.4426950408889634


def _fp_body(a1_ref, p2_ref, h2w_ref, h2b_ref, w1c_ref, w2c_ref, o_ref):
    a1 = a1_ref[...]
    v2 = p2_ref[0] + p2_ref[1] + a1
    hw = h2w_ref[...]
    z2 = (hw[:, 0:1] * v2[0:1, :] + hw[:, 1:2] * v2[1:2, :]) + h2b_ref[...]
    a2 = 1.0 / (1.0 + jnp.exp(-z2))

    ones = jnp.ones((1, FP), jnp.float32)

    def _soft(a, wc_ref):
        wc = wc_ref[...]
        l = (wc[:, 0:1] * a[0:1, :] + wc[:, 1:2] * a[1:2, :]
             + wc[:, 2:3])
        e = jnp.exp2(l)
        s = jnp.dot(ones, e, preferred_element_type=jnp.float32)
        return e * (1.0 / s)

    o_ref[...] = _soft(a1, w1c_ref) + _soft(a2, w2c_ref)


def _fingerprint(a1, p2, H2_w, H2_b, W1_w, W1_b, W2_w, W2_b):
    w1c = jnp.concatenate([W1_w, W1_b.reshape(FP, 1)], axis=1) * LOG2E
    w2c = jnp.concatenate([W2_w, W2_b.reshape(FP, 1)], axis=1) * LOG2E
    return pl.pallas_call(
        _fp_body,
        grid=(N_PAD // R,),
        in_specs=[
            pl.BlockSpec((2, R), lambda i: (0, i)),
            pl.BlockSpec((2, 2, R), lambda i: (0, 0, i)),
            pl.BlockSpec((2, 2), lambda i: (0, 0)),
            pl.BlockSpec((2, 1), lambda i: (0, 0)),
            pl.BlockSpec((FP, 3), lambda i: (0, 0)),
            pl.BlockSpec((FP, 3), lambda i: (0, 0)),
        ],
        out_specs=pl.BlockSpec((FP, R), lambda i: (0, i)),
        out_shape=jax.ShapeDtypeStruct((FP, N), jnp.float32),
    )(a1, p2, H2_w, H2_b.reshape(2, 1), w1c, w2c)


def kernel(x, edge_index, H1_w, H1_b, W1_w, W1_b, H2_w, H2_b, W2_w, W2_b):
    ei = edge_index.astype(jnp.int32)
    eif = ei.reshape(-1)
    dst2d = ei[1].reshape(E // 128, 128)
    xx = jnp.zeros((2, N_PAD), jnp.float32).at[:, :N].set(x.T)
    zz = jnp.zeros((SLICE,), jnp.float32)

    p1 = _segment_sum_sc(xx.reshape(-1), eif, dst2d, zz).reshape(2, 2, N_PAD)
    a1 = _affine_sigmoid(p1, xx, H1_w, H1_b)
    p2 = _segment_sum_sc(a1.reshape(-1), eif, dst2d, zz).reshape(2, 2, N_PAD)

    return _fingerprint(a1, p2, H2_w, H2_b, W1_w, W1_b, W2_w, W2_b).T


# speedup vs baseline: 1.0909x; 1.0642x over previous
import functools

import jax
import jax.numpy as jnp
from jax import lax
from jax.experimental import pallas as pl
from jax.experimental.pallas import tpu as pltpu
from jax.experimental.pallas import tpu_sc as plsc

N = 50000
FP = 1778
E = 3200000

NW = 32
CHUNK = 2048
CROWS = CHUNK // 128
NCH = E // CHUNK
CPT = NCH // NW
XTRA = NCH - CPT * NW
REM = NCH * CHUNK
RROWS = (E - REM) // 128
N_PAD = 50176
SLICE = N_PAD // 16

_mesh = plsc.VectorSubcoreMesh(core_axis_name="c", subcore_axis_name="s")


@functools.partial(
    pl.kernel,
    mesh=_mesh,
    out_type=jax.ShapeDtypeStruct((4 * N_PAD,), jnp.float32),
    compiler_params=pltpu.CompilerParams(needs_layout_passes=False),
    scratch_types=[
        pltpu.VMEM((2 * N_PAD,), jnp.float32),
        pltpu.VMEM((2, CHUNK), jnp.int32),
        pltpu.VMEM((2, CROWS, 128), jnp.int32),
        pltpu.VMEM((2, CHUNK), jnp.float32),
        pltpu.VMEM((2, CHUNK), jnp.float32),
        pltpu.VMEM((SLICE,), jnp.float32),
        pltpu.VMEM_SHARED((N_PAD,), jnp.float32),
        pltpu.VMEM_SHARED((N_PAD,), jnp.float32),
        pltpu.SemaphoreType.DMA,
        pltpu.SemaphoreType.DMA,
    ],
)
def _segment_sum_sc(xx, srch, dsth, zz, out, xxv, srcv, dstv, v0, v1, zbv,
                    acc0, acc1, sem_i, sem_s):
    cid = lax.axis_index("c")
    sid = lax.axis_index("s")
    wid = sid * 2 + cid

    pltpu.sync_copy(xx, xxv)

    pltpu.sync_copy(zz, zbv)
    pltpu.sync_copy(zbv, acc0.at[pl.ds(sid * SLICE, SLICE)])
    pltpu.sync_copy(zbv, acc1.at[pl.ds(sid * SLICE, SLICE)])
    plsc.subcore_barrier()

    nch = jnp.where(wid < XTRA, CPT + 1, CPT)
    ch0 = wid * CPT + jnp.minimum(wid, XTRA)

    def _gather(slot, n16):
        def _g(i, c2):
            s16 = srcv[slot, pl.ds(i * 16, 16)]
            v0[slot, pl.ds(i * 16, 16)] = plsc.load_gather(xxv, [s16])
            v1[slot, pl.ds(i * 16, 16)] = plsc.load_gather(xxv,
                                                           [s16 + N_PAD])
            return c2
        lax.fori_loop(0, n16, _g, 0)

    def _drain_scatters():
        for _ in range(4):
            pltpu.make_async_copy(xx.at[pl.ds(0, 1024)],
                                  zbv.at[pl.ds(0, 1024)], sem_s).wait()

    for _ in range(4):
        pltpu.async_copy(xx.at[pl.ds(0, 1024)], zbv.at[pl.ds(0, 1024)],
                         sem_s)
    pltpu.async_copy(srch.at[pl.ds(ch0 * CHUNK, CHUNK)], srcv.at[0], sem_i)
    pltpu.async_copy(dsth.at[pl.ds(ch0 * CROWS, CROWS)], dstv.at[0], sem_i)

    def _chunk(k, carry):
        p = lax.rem(k, 2)
        q = 1 - p
        _drain_scatters()
        pltpu.make_async_copy(srch.at[pl.ds(0, CHUNK)], srcv.at[p],
                              sem_i).wait()
        pltpu.make_async_copy(dsth.at[pl.ds(0, CROWS)], dstv.at[p],
                              sem_i).wait()
        c2 = jnp.minimum(ch0 + k + 1, NCH - 1)
        pltpu.async_copy(srch.at[pl.ds(c2 * CHUNK, CHUNK)], srcv.at[q],
                         sem_i)
        pltpu.async_copy(dsth.at[pl.ds(c2 * CROWS, CROWS)], dstv.at[q],
                         sem_i)
        _gather(p, CHUNK // 16)
        for j in range(CROWS):
            pltpu.async_copy(v0.at[p].at[pl.ds(j * 128, 128)],
                             acc0.at[dstv.at[p, j]], sem_s, add=True)
            pltpu.async_copy(v1.at[p].at[pl.ds(j * 128, 128)],
                             acc1.at[dstv.at[p, j]], sem_s, add=True)
        return carry
    lax.fori_loop(0, nch, _chunk, 0)

    _drain_scatters()
    pltpu.make_async_copy(srch.at[pl.ds(0, CHUNK)], srcv.at[0], sem_i).wait()
    pltpu.make_async_copy(dsth.at[pl.ds(0, CROWS)], dstv.at[0], sem_i).wait()

    @pl.when(wid == NW - 1)
    def _rem():
        pltpu.sync_copy(srch.at[pl.ds(REM, 128 * RROWS)],
                        srcv.at[0].at[pl.ds(0, 128 * RROWS)])
        pltpu.sync_copy(dsth.at[pl.ds(NCH * CROWS, RROWS)],
                        dstv.at[0].at[pl.ds(0, RROWS)])
        _gather(0, (128 * RROWS) // 16)
        for j in range(RROWS):
            pltpu.sync_copy(v0.at[0].at[pl.ds(j * 128, 128)],
                            acc0.at[dstv.at[0, j]], add=True)
            pltpu.sync_copy(v1.at[0].at[pl.ds(j * 128, 128)],
                            acc1.at[dstv.at[0, j]], add=True)

    plsc.subcore_barrier()
    pltpu.sync_copy(acc0.at[pl.ds(sid * SLICE, SLICE)], zbv)
    pltpu.sync_copy(zbv, out.at[pl.ds(cid * 2 * N_PAD + sid * SLICE, SLICE)])
    pltpu.sync_copy(acc1.at[pl.ds(sid * SLICE, SLICE)], zbv)
    pltpu.sync_copy(zbv, out.at[pl.ds((cid * 2 + 1) * N_PAD + sid * SLICE,
                                      SLICE)])


def _affine_body(p_ref, xx_ref, hw_ref, hb_ref, o_ref):
    v = p_ref[0] + p_ref[1] + xx_ref[...]
    hw = hw_ref[...]
    z = (hw[:, 0:1] * v[0:1, :] + hw[:, 1:2] * v[1:2, :]) + hb_ref[...]
    o_ref[...] = 1.0 / (1.0 + jnp.exp(-z))


def _affine_sigmoid(p, xx, H_w, H_b):
    return pl.pallas_call(
        _affine_body,
        out_shape=jax.ShapeDtypeStruct((2, N_PAD), jnp.float32),
    )(p, xx, H_w, H_b.reshape(2, 1))


R = 1792
LOG2E = 1.4426950408889634


def _fp_body(a1_ref, p2_ref, h2w_ref, h2b_ref, w1c_ref, w2c_ref, o_ref):
    a1 = a1_ref[...]
    v2 = p2_ref[0] + p2_ref[1] + a1
    hw = h2w_ref[...]
    z2 = (hw[:, 0:1] * v2[0:1, :] + hw[:, 1:2] * v2[1:2, :]) + h2b_ref[...]
    a2 = 1.0 / (1.0 + jnp.exp(-z2))

    ones = jnp.ones((1, FP), jnp.float32)

    def _soft(a, wc_ref):
        wc = wc_ref[...]
        l = (wc[:, 0:1] * a[0:1, :] + wc[:, 1:2] * a[1:2, :]
             + wc[:, 2:3])
        e = jnp.exp2(l)
        s = jnp.dot(ones, e, preferred_element_type=jnp.float32)
        return e * (1.0 / s)

    o_ref[...] = _soft(a1, w1c_ref) + _soft(a2, w2c_ref)


def _fingerprint(a1, p2, H2_w, H2_b, W1_w, W1_b, W2_w, W2_b):
    w1c = jnp.concatenate([W1_w, W1_b.reshape(FP, 1)], axis=1) * LOG2E
    w2c = jnp.concatenate([W2_w, W2_b.reshape(FP, 1)], axis=1) * LOG2E
    return pl.pallas_call(
        _fp_body,
        grid=(N_PAD // R,),
        in_specs=[
            pl.BlockSpec((2, R), lambda i: (0, i)),
            pl.BlockSpec((2, 2, R), lambda i: (0, 0, i)),
            pl.BlockSpec((2, 2), lambda i: (0, 0)),
            pl.BlockSpec((2, 1), lambda i: (0, 0)),
            pl.BlockSpec((FP, 3), lambda i: (0, 0)),
            pl.BlockSpec((FP, 3), lambda i: (0, 0)),
        ],
        out_specs=pl.BlockSpec((FP, R), lambda i: (0, i)),
        out_shape=jax.ShapeDtypeStruct((FP, N), jnp.float32),
    )(a1, p2, H2_w, H2_b.reshape(2, 1), w1c, w2c)


def kernel(x, edge_index, H1_w, H1_b, W1_w, W1_b, H2_w, H2_b, W2_w, W2_b):
    ei = edge_index.astype(jnp.int32)
    eif = ei.reshape(-1)
    dst2d = ei[1].reshape(E // 128, 128)
    xx = jnp.zeros((2, N_PAD), jnp.float32).at[:, :N].set(x.T)
    zz = jnp.zeros((SLICE,), jnp.float32)

    p1 = _segment_sum_sc(xx.reshape(-1), eif, dst2d, zz).reshape(2, 2, N_PAD)
    a1 = _affine_sigmoid(p1, xx, H1_w, H1_b)
    p2 = _segment_sum_sc(a1.reshape(-1), eif, dst2d, zz).reshape(2, 2, N_PAD)

    return _fingerprint(a1, p2, H2_w, H2_b, W1_w, W1_b, W2_w, W2_b).T
